# baseline (speedup 1.0000x reference)
import jax
import jax.numpy as jnp
from jax import lax
from jax.experimental import pallas as pl
from jax.experimental.pallas import tpu as pltpu
from jax.experimental.pallas import tpu_sc as plsc

_NUM_CLASSES = 90
_C_PAD = 96
_FEA = 128
_BATCH = 16384
_MARGIN = 1.0
_NEG_BIG = -1e30

_NC, _NS = 2, 16
_NW = _NC * _NS
_ROWS_PER_W = _BATCH // _NW
_GROUPS = _ROWS_PER_W // 16

_B_BLK = 2048


def _tc_dots_kernel(x_ref, c_ref, out_ref):
    c = c_ref[...]
    nrm = jnp.sqrt(jnp.sum(c * c, axis=1, keepdims=True))
    nc = c / (nrm + 1e-12)
    d = lax.dot_general(nc, x_ref[...], (((1,), (1,)), ((), ())),
                        preferred_element_type=jnp.float32)
    row = lax.broadcasted_iota(jnp.int32, d.shape, 0)
    out_ref[...] = jnp.where(row < _NUM_CLASSES, d, _NEG_BIG)


def _sc_loss_kernel(dots_hbm, lab_hbm, out_hbm, dots_v, lab_v, acc_v):
    wid = lax.axis_index("s") * _NC + lax.axis_index("c")
    base = wid * _ROWS_PER_W
    pltpu.sync_copy(dots_hbm.at[:, pl.ds(base, _ROWS_PER_W)], dots_v)
    pltpu.sync_copy(lab_hbm.at[pl.ds(base, _ROWS_PER_W)], lab_v)

    def body(g, acc):
        off = g * 16
        labv = lab_v[pl.ds(off, 16)]
        m = jnp.full((16,), _NEG_BIG, jnp.float32)
        p = jnp.zeros((16,), jnp.float32)
        for j in range(_C_PAD):
            v = dots_v[j, pl.ds(off, 16)]
            own = labv == j
            m = jnp.maximum(m, jnp.where(own, _NEG_BIG, v))
            p = jnp.where(own, v, p)
        return acc + jnp.maximum(_MARGIN + m - p, 0.0)

    acc = lax.fori_loop(0, _GROUPS, body, jnp.zeros((16,), jnp.float32))
    acc_v[...] = acc
    pltpu.sync_copy(acc_v, out_hbm.at[wid])


def kernel(x, labels, centers):
    labels = labels.astype(jnp.int32)
    cpad = jnp.pad(centers, ((0, _C_PAD - _NUM_CLASSES), (0, 0)))

    dots_t = pl.pallas_call(
        _tc_dots_kernel,
        grid=(_BATCH // _B_BLK,),
        in_specs=[
            pl.BlockSpec((_B_BLK, _FEA), lambda i: (i, 0)),
            pl.BlockSpec((_C_PAD, _FEA), lambda i: (0, 0)),
        ],
        out_specs=pl.BlockSpec((_C_PAD, _B_BLK), lambda i: (0, i)),
        out_shape=jax.ShapeDtypeStruct((_C_PAD, _BATCH), jnp.float32),
    )(x, cpad)

    partials = pl.kernel(
        _sc_loss_kernel,
        out_type=jax.ShapeDtypeStruct((_NW, 16), jnp.float32),
        mesh=plsc.VectorSubcoreMesh(core_axis_name="c", subcore_axis_name="s"),
        scratch_types=[
            pltpu.VMEM((_C_PAD, _ROWS_PER_W), jnp.float32),
            pltpu.VMEM((_ROWS_PER_W,), jnp.int32),
            pltpu.VMEM((16,), jnp.float32),
        ],
    )(dots_t, labels)

    return jnp.sum(partials) / _BATCH
